# baseline (speedup 1.0000x reference)
import functools

import jax
import jax.numpy as jnp
import numpy as np
from jax import lax
from jax.experimental import pallas as pl
from jax.experimental.pallas import tpu as pltpu
from jax.experimental.pallas import tpu_sc as plsc

NC = 2
NS = 16
LN = 16
CH = 128

EPS = 1e-5
F32 = jnp.float32


def _silu(v):
    return v / (1.0 + jnp.exp(-v))


def _leaky(v):
    return jnp.where(v >= 0, v, 0.01 * v)


def _gather_pos_body(posw_h, si0_h, di0_h, si1_h, di1_h,
                     ps0_h, pd0_h, ps1_h, pd1_h,
                     idx, rows, sem, *, nchunks):
    c = lax.axis_index("c")
    t = lax.axis_index("s")
    row0 = t * nchunks

    def run(i_h, o_h):
        def chunk(k, carry):
            pltpu.sync_copy(i_h.at[row0 + k], idx)
            pltpu.async_copy(posw_h.at[idx], rows, sem).wait()
            pltpu.sync_copy(rows, o_h.at[pl.ds((row0 + k) * CH, CH)])
            return carry

        lax.fori_loop(0, nchunks, chunk, 0)

    @pl.when(c == 0)
    def _():
        run(si0_h, ps0_h)
        run(di0_h, pd0_h)

    @pl.when(c == 1)
    def _():
        run(si1_h, ps1_h)
        run(di1_h, pd1_h)


def _agg_body(h_h, r0_h, r1_h, si0_h, di0_h, si1_h, di1_h, out_h,
              sit, dit, hbuf, rbuf, semh, sems, acc,
              *, nchunks, stripe, hdim):
    c = lax.axis_index("c")
    t = lax.axis_index("s")

    def zrow(i, carry):
        for j in range(hdim // LN):
            rbuf[i, pl.ds(j * LN, LN)] = jnp.zeros((LN,), F32)
        return carry

    lax.fori_loop(0, CH, zrow, 0)
    for q in range(stripe // CH):
        pltpu.sync_copy(rbuf, acc.at[pl.ds(t * stripe + q * CH, CH)])
    plsc.subcore_barrier()

    row0 = t * nchunks

    def run(r_h, si_h, di_h):
        @pl.loop(0, nchunks // 8)
        def _(g):
            pltpu.sync_copy(si_h.at[pl.ds(row0 + g * 8, 8)], sit)
            pltpu.sync_copy(di_h.at[pl.ds(row0 + g * 8, 8)], dit)
            for b in range(8):
                pltpu.async_copy(h_h.at[sit.at[b]], hbuf, semh).wait()
                pltpu.sync_copy(
                    r_h.at[pl.ds((row0 + g * 8 + b) * CH, CH)], rbuf)

                def mrow(i, cc):
                    for j in range(hdim // LN):
                        hbuf[i, pl.ds(j * LN, LN)] = (
                            hbuf[i, pl.ds(j * LN, LN)]
                            * rbuf[i, pl.ds(j * LN, LN)])
                    return cc

                lax.fori_loop(0, CH, mrow, 0)
                pltpu.async_copy(hbuf, acc.at[dit.at[b]], sems, add=True)
                pltpu.make_async_copy(h_h.at[pl.ds(0, CH)], hbuf, sems).wait()

    @pl.when(c == 0)
    def _():
        run(r0_h, si0_h, di0_h)

    @pl.when(c == 1)
    def _():
        run(r1_h, si1_h, di1_h)

    plsc.subcore_barrier()
    for q in range(stripe // CH):
        r0 = t * stripe + q * CH
        pltpu.sync_copy(acc.at[pl.ds(r0, CH)], hbuf)
        pltpu.sync_copy(hbuf, out_h.at[c, pl.ds(r0, CH)])


def _in_mlp_body(x_ref, w_ref, b_ref, o_ref):
    v = jnp.dot(x_ref[...], w_ref[...], preferred_element_type=F32) + b_ref[...]
    o_ref[...] = _silu(v)


def _radial_body(ps_ref, pd_ref, w_ref, b_ref, o0, o1, o2, *, inv_sigma):
    df = ps_ref[...] - pd_ref[...]
    d = jnp.sqrt(jnp.sum(df * df, axis=1, keepdims=True))
    mu = lax.broadcasted_iota(jnp.int32, (1, 9), 1).astype(F32) * 0.75
    dm = (d - mu) * inv_sigma
    r = jnp.exp(-(dm * dm))
    outs = (o0, o1, o2)
    for l in range(3):
        v = jnp.dot(r, w_ref[l], preferred_element_type=F32) + b_ref[l]
        outs[l][...] = _silu(v)


def _node_mlp_body(h_ref, a_ref, b_ref, w1, b1, g1, e1, w2, b2, g2, e2, o_ref,
                   *, res):
    hh = h_ref[...]
    va = jnp.dot(hh + a_ref[0], w1[...], preferred_element_type=F32) + b1[...]
    va = _leaky(va) * g1[...] + e1[...]
    vb = jnp.dot(hh + b_ref[0], w2[...], preferred_element_type=F32) + b2[...]
    vb = _leaky(vb) * g2[...] + e2[...]
    o_ref[...] = va + vb + hh if res else va + vb


def _head_body(h_ref, bat_ref, fw, fb, fg, fe, wo, bo, z_ref, p_ref, *,
               nsteps, gdim):
    i = pl.program_id(0)

    @pl.when(i == 0)
    def _():
        z_ref[...] = jnp.zeros_like(z_ref)

    seg = lax.broadcasted_iota(jnp.int32, (gdim, bat_ref.shape[2]), 0)
    m = (bat_ref[0] == seg).astype(F32)
    z_ref[...] += jnp.dot(m, h_ref[...], preferred_element_type=F32)

    @pl.when(i == nsteps - 1)
    def _():
        z = z_ref[...]
        for l in range(3):
            v = jnp.dot(z, fw[l], preferred_element_type=F32) + fb[l]
            z = _leaky(v) * fg[l] + fe[l]
        z_ref[...] = z
        p_ref[...] = jnp.dot(z, wo[...], preferred_element_type=F32) + bo[...]


def kernel(x, edge_index_intra, edge_index_inter, pos, y, batch, params):
    N, D = x.shape
    E = edge_index_intra.shape[1]
    H = params['lin_node']['W'].shape[1]
    G = y.shape[0]

    NPAD = ((N + 1 + NS * CH - 1) // (NS * CH)) * (NS * CH)
    EPAD = ((E + NS * CH * 8 - 1) // (NS * CH * 8)) * (NS * CH * 8)
    ept = EPAD // NS
    nchunks = ept // CH
    stripe = NPAD // NS
    BR = NPAD // 8
    npad, epad = NPAD - N, EPAD - E

    xp = jnp.concatenate([x, jnp.zeros((npad, D), F32)], 0)
    posw = jnp.pad(pos, ((0, npad), (0, 125)))

    spare = NPAD - N
    ar = jnp.arange(epad, dtype=jnp.int32)
    pad_src = (ar * 97) % N
    pad_dst = N + (ar % spare)

    def padidx(ei):
        return (jnp.concatenate([ei[0], pad_src]).reshape(-1, CH),
                jnp.concatenate([ei[1], pad_dst]).reshape(-1, CH))

    si0, di0 = padidx(edge_index_intra)
    si1, di1 = padidx(edge_index_inter)

    mesh = plsc.VectorSubcoreMesh(core_axis_name="c", subcore_axis_name="s",
                                  num_cores=NC, num_subcores=NS)

    gp_call = pl.kernel(
        functools.partial(_gather_pos_body, nchunks=nchunks),
        out_type=[jax.ShapeDtypeStruct((EPAD, 128), F32)] * 4,
        mesh=mesh,
        scratch_types=[pltpu.VMEM((CH,), jnp.int32),
                       pltpu.VMEM((CH, 128), F32),
                       pltpu.SemaphoreType.DMA],
    )
    ps0, pd0, ps1, pd1 = gp_call(posw, si0, di0, si1, di1)

    h = pl.pallas_call(
        _in_mlp_body,
        grid=(8,),
        in_specs=[pl.BlockSpec((BR, D), lambda i: (i, 0)),
                  pl.BlockSpec((D, H), lambda i: (0, 0)),
                  pl.BlockSpec((1, H), lambda i: (0, 0))],
        out_specs=pl.BlockSpec((BR, H), lambda i: (i, 0)),
        out_shape=jax.ShapeDtypeStruct((NPAD, H), F32),
    )(xp, params['lin_node']['W'], params['lin_node']['b'].reshape(1, H))

    ER = 2048
    EG = EPAD // ER
    inv_sigma = float(9.0 / 6.0)
    gl = params['gconv']

    def radial_call(ps, pd, key):
        wst = jnp.stack([gl[l][key]['W'] for l in range(3)])
        bst = jnp.stack([gl[l][key]['b'].reshape(1, H) for l in range(3)])
        return pl.pallas_call(
            functools.partial(_radial_body, inv_sigma=inv_sigma),
            grid=(EG,),
            in_specs=[pl.BlockSpec((ER, 128), lambda e: (e, 0)),
                      pl.BlockSpec((ER, 128), lambda e: (e, 0)),
                      pl.BlockSpec((3, 9, H), lambda e: (0, 0, 0)),
                      pl.BlockSpec((3, 1, H), lambda e: (0, 0, 0))],
            out_specs=[pl.BlockSpec((ER, H), lambda e: (e, 0))] * 3,
            out_shape=[jax.ShapeDtypeStruct((EPAD, H), F32)] * 3,
        )(ps, pd, wst, bst)

    rad0 = radial_call(ps0, pd0, 'coord_cov')
    rad1 = radial_call(ps1, pd1, 'coord_ncov')

    agg_call = pl.kernel(
        functools.partial(_agg_body, nchunks=nchunks, stripe=stripe, hdim=H),
        out_type=jax.ShapeDtypeStruct((2, NPAD, H), F32),
        mesh=mesh,
        scratch_types=[pltpu.VMEM((8, CH), jnp.int32)] * 2
        + [pltpu.VMEM((CH, H), F32)] * 2
        + [pltpu.SemaphoreType.DMA] * 2
        + [pltpu.VMEM_SHARED((NPAD, H), F32)],
    )

    bn_scale = 1.0 / np.sqrt(1.0 + EPS)

    def node_mlp(hcur, agg, lp, res):
        args = [hcur, agg, agg]
        for key in ('node_cov', 'node_ncov'):
            p = lp[key]
            args += [p['W'], p['b'].reshape(1, H),
                     (p['g'] * bn_scale).reshape(1, H), p['be'].reshape(1, H)]
        wspec = [pl.BlockSpec((H, H), lambda i: (0, 0)),
                 pl.BlockSpec((1, H), lambda i: (0, 0)),
                 pl.BlockSpec((1, H), lambda i: (0, 0)),
                 pl.BlockSpec((1, H), lambda i: (0, 0))]
        return pl.pallas_call(
            functools.partial(_node_mlp_body, res=res),
            grid=(8,),
            in_specs=[pl.BlockSpec((BR, H), lambda i: (i, 0)),
                      pl.BlockSpec((1, BR, H), lambda i: (0, i, 0)),
                      pl.BlockSpec((1, BR, H), lambda i: (1, i, 0))]
            + wspec + wspec,
            out_specs=pl.BlockSpec((BR, H), lambda i: (i, 0)),
            out_shape=jax.ShapeDtypeStruct((NPAD, H), F32),
        )(*args)

    for l in range(3):
        agg = agg_call(h, rad0[l], rad1[l], si0, di0, si1, di1)
        h = node_mlp(h, agg, gl[l], res=(l > 0))

    batp = jnp.concatenate(
        [batch.astype(jnp.int32), jnp.full((npad,), G, jnp.int32)]).reshape(8, 1, BR)
    fw = jnp.stack([params['fc'][l]['W'] for l in range(3)])
    fb = jnp.stack([params['fc'][l]['b'].reshape(1, H) for l in range(3)])
    fg = jnp.stack([(params['fc'][l]['g'] * bn_scale).reshape(1, H)
                    for l in range(3)])
    fe = jnp.stack([params['fc'][l]['be'].reshape(1, H) for l in range(3)])

    z, pred = pl.pallas_call(
        functools.partial(_head_body, nsteps=8, gdim=G),
        grid=(8,),
        in_specs=[pl.BlockSpec((BR, H), lambda i: (i, 0)),
                  pl.BlockSpec((1, 1, BR), lambda i: (i, 0, 0)),
                  pl.BlockSpec((3, H, H), lambda i: (0, 0, 0)),
                  pl.BlockSpec((3, 1, H), lambda i: (0, 0, 0)),
                  pl.BlockSpec((3, 1, H), lambda i: (0, 0, 0)),
                  pl.BlockSpec((3, 1, H), lambda i: (0, 0, 0)),
                  pl.BlockSpec((H, 1), lambda i: (0, 0)),
                  pl.BlockSpec((1, 1), lambda i: (0, 0))],
        out_specs=[pl.BlockSpec((G, H), lambda i: (0, 0)),
                   pl.BlockSpec((G, 1), lambda i: (0, 0))],
        out_shape=[jax.ShapeDtypeStruct((G, H), F32),
                   jax.ShapeDtypeStruct((G, 1), F32)],
    )(h, batp, fw, fb, fg, fe, params['out']['W'],
      params['out']['b'].reshape(1, 1))

    return (pred.reshape(-1), y, z)


# speedup vs baseline: 1.0895x; 1.0895x over previous
import functools

import jax
import jax.numpy as jnp
import numpy as np
from jax import lax
from jax.experimental import pallas as pl
from jax.experimental.pallas import tpu as pltpu
from jax.experimental.pallas import tpu_sc as plsc

NC = 2
NS = 16
LN = 16
CH = 128

EPS = 1e-5
F32 = jnp.float32


def _silu(v):
    return v / (1.0 + jnp.exp(-v))


def _leaky(v):
    return jnp.where(v >= 0, v, 0.01 * v)


def _gather_pos_body(posw_h, si0_h, di0_h, si1_h, di1_h,
                     ps0_h, pd0_h, ps1_h, pd1_h,
                     idx, rows, sem, *, nchunks):
    c = lax.axis_index("c")
    t = lax.axis_index("s")
    row0 = t * nchunks

    def run(i_h, o_h):
        def chunk(k, carry):
            pltpu.sync_copy(i_h.at[row0 + k], idx)
            pltpu.async_copy(posw_h.at[idx], rows, sem).wait()
            pltpu.sync_copy(rows, o_h.at[pl.ds((row0 + k) * CH, CH)])
            return carry

        lax.fori_loop(0, nchunks, chunk, 0)

    @pl.when(c == 0)
    def _():
        run(si0_h, ps0_h)
        run(di0_h, pd0_h)

    @pl.when(c == 1)
    def _():
        run(si1_h, ps1_h)
        run(di1_h, pd1_h)


def _agg_body(h_h, r0_h, r1_h, si0_h, di0_h, si1_h, di1_h, out_h,
              sit, dit, hbuf, rbuf, semh, semr, sems, acc,
              *, nchunks, stripe, hdim):
    c = lax.axis_index("c")
    t = lax.axis_index("s")

    def zrow(i, carry):
        for j in range(hdim // LN):
            rbuf[i, pl.ds(j * LN, LN)] = jnp.zeros((LN,), F32)
        return carry

    lax.fori_loop(0, CH, zrow, 0)
    for q in range(stripe // CH):
        pltpu.sync_copy(rbuf, acc.at[pl.ds(t * stripe + q * CH, CH)])
    plsc.subcore_barrier()

    row0 = t * nchunks

    def run(r_h, si_h, di_h):
        @pl.loop(0, nchunks // 8)
        def _(g):
            pltpu.sync_copy(si_h.at[pl.ds(row0 + g * 8, 8)], sit)
            pltpu.sync_copy(di_h.at[pl.ds(row0 + g * 8, 8)], dit)
            for b in range(8):
                pltpu.async_copy(h_h.at[sit.at[b]], hbuf, semh)
                pltpu.async_copy(
                    r_h.at[pl.ds((row0 + g * 8 + b) * CH, CH)], rbuf, semr)
                pltpu.make_async_copy(h_h.at[pl.ds(0, CH)], hbuf, semh).wait()
                pltpu.make_async_copy(r_h.at[pl.ds(0, CH)], rbuf, semr).wait()

                def mrow(i, cc):
                    for j in range(hdim // LN):
                        hbuf[i, pl.ds(j * LN, LN)] = (
                            hbuf[i, pl.ds(j * LN, LN)]
                            * rbuf[i, pl.ds(j * LN, LN)])
                    return cc

                lax.fori_loop(0, CH, mrow, 0)
                pltpu.async_copy(hbuf, acc.at[dit.at[b]], sems, add=True)
                pltpu.make_async_copy(h_h.at[pl.ds(0, CH)], hbuf, sems).wait()

    @pl.when(c == 0)
    def _():
        run(r0_h, si0_h, di0_h)

    @pl.when(c == 1)
    def _():
        run(r1_h, si1_h, di1_h)

    plsc.subcore_barrier()
    for q in range(stripe // CH):
        r0 = t * stripe + q * CH
        pltpu.sync_copy(acc.at[pl.ds(r0, CH)], hbuf)
        pltpu.sync_copy(hbuf, out_h.at[c, pl.ds(r0, CH)])


def _in_mlp_body(x_ref, w_ref, b_ref, o_ref):
    v = jnp.dot(x_ref[...], w_ref[...], preferred_element_type=F32) + b_ref[...]
    o_ref[...] = _silu(v)


def _radial_body(ps_ref, pd_ref, w_ref, b_ref, o0, o1, o2, *, inv_sigma):
    df = ps_ref[...] - pd_ref[...]
    d = jnp.sqrt(jnp.sum(df * df, axis=1, keepdims=True))
    mu = lax.broadcasted_iota(jnp.int32, (1, 9), 1).astype(F32) * 0.75
    dm = (d - mu) * inv_sigma
    r = jnp.exp(-(dm * dm))
    outs = (o0, o1, o2)
    for l in range(3):
        v = jnp.dot(r, w_ref[l], preferred_element_type=F32) + b_ref[l]
        outs[l][...] = _silu(v)


def _node_mlp_body(h_ref, a_ref, b_ref, w1, b1, g1, e1, w2, b2, g2, e2, o_ref,
                   *, res):
    hh = h_ref[...]
    va = jnp.dot(hh + a_ref[0], w1[...], preferred_element_type=F32) + b1[...]
    va = _leaky(va) * g1[...] + e1[...]
    vb = jnp.dot(hh + b_ref[0], w2[...], preferred_element_type=F32) + b2[...]
    vb = _leaky(vb) * g2[...] + e2[...]
    o_ref[...] = va + vb + hh if res else va + vb


def _head_body(h_ref, bat_ref, fw, fb, fg, fe, wo, bo, z_ref, p_ref, *,
               nsteps, gdim):
    i = pl.program_id(0)

    @pl.when(i == 0)
    def _():
        z_ref[...] = jnp.zeros_like(z_ref)

    seg = lax.broadcasted_iota(jnp.int32, (gdim, bat_ref.shape[2]), 0)
    m = (bat_ref[0] == seg).astype(F32)
    z_ref[...] += jnp.dot(m, h_ref[...], preferred_element_type=F32)

    @pl.when(i == nsteps - 1)
    def _():
        z = z_ref[...]
        for l in range(3):
            v = jnp.dot(z, fw[l], preferred_element_type=F32) + fb[l]
            z = _leaky(v) * fg[l] + fe[l]
        z_ref[...] = z
        p_ref[...] = jnp.dot(z, wo[...], preferred_element_type=F32) + bo[...]


def kernel(x, edge_index_intra, edge_index_inter, pos, y, batch, params):
    N, D = x.shape
    E = edge_index_intra.shape[1]
    H = params['lin_node']['W'].shape[1]
    G = y.shape[0]

    NPAD = ((N + 1 + NS * CH - 1) // (NS * CH)) * (NS * CH)
    EPAD = ((E + NS * CH * 8 - 1) // (NS * CH * 8)) * (NS * CH * 8)
    ept = EPAD // NS
    nchunks = ept // CH
    stripe = NPAD // NS
    BR = NPAD // 8
    npad, epad = NPAD - N, EPAD - E

    xp = jnp.concatenate([x, jnp.zeros((npad, D), F32)], 0)
    posw = jnp.pad(pos, ((0, npad), (0, 125)))

    spare = NPAD - N
    ar = jnp.arange(epad, dtype=jnp.int32)
    pad_src = (ar * 97) % N
    pad_dst = N + (ar % spare)

    def padidx(ei):
        return (jnp.concatenate([ei[0], pad_src]).reshape(-1, CH),
                jnp.concatenate([ei[1], pad_dst]).reshape(-1, CH))

    si0, di0 = padidx(edge_index_intra)
    si1, di1 = padidx(edge_index_inter)

    mesh = plsc.VectorSubcoreMesh(core_axis_name="c", subcore_axis_name="s",
                                  num_cores=NC, num_subcores=NS)

    gp_call = pl.kernel(
        functools.partial(_gather_pos_body, nchunks=nchunks),
        out_type=[jax.ShapeDtypeStruct((EPAD, 128), F32)] * 4,
        mesh=mesh,
        scratch_types=[pltpu.VMEM((CH,), jnp.int32),
                       pltpu.VMEM((CH, 128), F32),
                       pltpu.SemaphoreType.DMA],
    )
    ps0, pd0, ps1, pd1 = gp_call(posw, si0, di0, si1, di1)

    h = pl.pallas_call(
        _in_mlp_body,
        grid=(8,),
        in_specs=[pl.BlockSpec((BR, D), lambda i: (i, 0)),
                  pl.BlockSpec((D, H), lambda i: (0, 0)),
                  pl.BlockSpec((1, H), lambda i: (0, 0))],
        out_specs=pl.BlockSpec((BR, H), lambda i: (i, 0)),
        out_shape=jax.ShapeDtypeStruct((NPAD, H), F32),
    )(xp, params['lin_node']['W'], params['lin_node']['b'].reshape(1, H))

    ER = 2048
    EG = EPAD // ER
    inv_sigma = float(9.0 / 6.0)
    gl = params['gconv']

    def radial_call(ps, pd, key):
        wst = jnp.stack([gl[l][key]['W'] for l in range(3)])
        bst = jnp.stack([gl[l][key]['b'].reshape(1, H) for l in range(3)])
        return pl.pallas_call(
            functools.partial(_radial_body, inv_sigma=inv_sigma),
            grid=(EG,),
            in_specs=[pl.BlockSpec((ER, 128), lambda e: (e, 0)),
                      pl.BlockSpec((ER, 128), lambda e: (e, 0)),
                      pl.BlockSpec((3, 9, H), lambda e: (0, 0, 0)),
                      pl.BlockSpec((3, 1, H), lambda e: (0, 0, 0))],
            out_specs=[pl.BlockSpec((ER, H), lambda e: (e, 0))] * 3,
            out_shape=[jax.ShapeDtypeStruct((EPAD, H), F32)] * 3,
        )(ps, pd, wst, bst)

    rad0 = radial_call(ps0, pd0, 'coord_cov')
    rad1 = radial_call(ps1, pd1, 'coord_ncov')

    agg_call = pl.kernel(
        functools.partial(_agg_body, nchunks=nchunks, stripe=stripe, hdim=H),
        out_type=jax.ShapeDtypeStruct((2, NPAD, H), F32),
        mesh=mesh,
        scratch_types=[pltpu.VMEM((8, CH), jnp.int32)] * 2
        + [pltpu.VMEM((CH, H), F32)] * 2
        + [pltpu.SemaphoreType.DMA] * 3
        + [pltpu.VMEM_SHARED((NPAD, H), F32)],
    )

    bn_scale = 1.0 / np.sqrt(1.0 + EPS)

    def node_mlp(hcur, agg, lp, res):
        args = [hcur, agg, agg]
        for key in ('node_cov', 'node_ncov'):
            p = lp[key]
            args += [p['W'], p['b'].reshape(1, H),
                     (p['g'] * bn_scale).reshape(1, H), p['be'].reshape(1, H)]
        wspec = [pl.BlockSpec((H, H), lambda i: (0, 0)),
                 pl.BlockSpec((1, H), lambda i: (0, 0)),
                 pl.BlockSpec((1, H), lambda i: (0, 0)),
                 pl.BlockSpec((1, H), lambda i: (0, 0))]
        return pl.pallas_call(
            functools.partial(_node_mlp_body, res=res),
            grid=(8,),
            in_specs=[pl.BlockSpec((BR, H), lambda i: (i, 0)),
                      pl.BlockSpec((1, BR, H), lambda i: (0, i, 0)),
                      pl.BlockSpec((1, BR, H), lambda i: (1, i, 0))]
            + wspec + wspec,
            out_specs=pl.BlockSpec((BR, H), lambda i: (i, 0)),
            out_shape=jax.ShapeDtypeStruct((NPAD, H), F32),
        )(*args)

    for l in range(3):
        agg = agg_call(h, rad0[l], rad1[l], si0, di0, si1, di1)
        h = node_mlp(h, agg, gl[l], res=(l > 0))

    batp = jnp.concatenate(
        [batch.astype(jnp.int32), jnp.full((npad,), G, jnp.int32)]).reshape(8, 1, BR)
    fw = jnp.stack([params['fc'][l]['W'] for l in range(3)])
    fb = jnp.stack([params['fc'][l]['b'].reshape(1, H) for l in range(3)])
    fg = jnp.stack([(params['fc'][l]['g'] * bn_scale).reshape(1, H)
                    for l in range(3)])
    fe = jnp.stack([params['fc'][l]['be'].reshape(1, H) for l in range(3)])

    z, pred = pl.pallas_call(
        functools.partial(_head_body, nsteps=8, gdim=G),
        grid=(8,),
        in_specs=[pl.BlockSpec((BR, H), lambda i: (i, 0)),
                  pl.BlockSpec((1, 1, BR), lambda i: (i, 0, 0)),
                  pl.BlockSpec((3, H, H), lambda i: (0, 0, 0)),
                  pl.BlockSpec((3, 1, H), lambda i: (0, 0, 0)),
                  pl.BlockSpec((3, 1, H), lambda i: (0, 0, 0)),
                  pl.BlockSpec((3, 1, H), lambda i: (0, 0, 0)),
                  pl.BlockSpec((H, 1), lambda i: (0, 0)),
                  pl.BlockSpec((1, 1), lambda i: (0, 0))],
        out_specs=[pl.BlockSpec((G, H), lambda i: (0, 0)),
                   pl.BlockSpec((G, 1), lambda i: (0, 0))],
        out_shape=[jax.ShapeDtypeStruct((G, H), F32),
                   jax.ShapeDtypeStruct((G, 1), F32)],
    )(h, batp, fw, fb, fg, fe, params['out']['W'],
      params['out']['b'].reshape(1, 1))

    return (pred.reshape(-1), y, z)
